# Initial kernel scaffold; baseline (speedup 1.0000x reference)
import jax
import jax.numpy as jnp
from jax.experimental import pallas as pl


def kernel(x, edge_index, W1, b1, W2, b2, W3, b3, Wl, bl):
    raise NotImplementedError("write your pallas kernel here")


# speedup vs baseline: 9.4731x; 9.4731x over previous
import functools

import jax
import jax.numpy as jnp
from jax import lax
from jax.experimental import pallas as pl
from jax.experimental.pallas import tpu as pltpu
from jax.experimental.pallas import tpu_sc as plsc

_N = 10000
_NPAD = 10240
_D = 128
_E = 320000
_NC = 2
_NS = 16
_NT = _NC * _NS
_KB = 128
_NBATCH = 79
_EPW = _KB * _NBATCH
_EPAD = _EPW * _NT
_RPT = _NPAD // _NS
_BR = 1024

_sc_mesh = plsc.VectorSubcoreMesh(core_axis_name="c", subcore_axis_name="s")


@functools.partial(
    pl.kernel,
    out_type=jax.ShapeDtypeStruct((_NC, _NPAD, _D), jnp.float32),
    mesh=_sc_mesh,
    scratch_types=[
        pltpu.VMEM((_KB,), jnp.int32),
        pltpu.VMEM((_KB, _D), jnp.float32),
        pltpu.VMEM_SHARED((_NPAD, _D), jnp.float32),
    ],
)
def _deg_kernel(dst_hbm, deg_hbm, idx_v, rows_v, deg_s):
    core = lax.axis_index("c")
    sub = lax.axis_index("s")
    tid = sub * _NC + core

    def _fill(val):
        def _f(i, c):
            rows_v[i // (_D // 16), pl.ds((i % (_D // 16)) * 16, 16)] = jnp.full(
                (16,), val, jnp.float32)
            return c
        lax.fori_loop(0, _KB * (_D // 16), _f, 0)

    _fill(0.0)

    def _zero(r, c):
        pltpu.sync_copy(rows_v, deg_s.at[pl.ds(sub * _RPT + r * _KB, _KB), :])
        return c

    lax.fori_loop(0, _RPT // _KB, _zero, 0)
    _fill(1.0)
    plsc.subcore_barrier()

    def _step(i, c):
        pltpu.sync_copy(dst_hbm.at[pl.ds(tid * _EPW + i * _KB, _KB)], idx_v)
        pltpu.sync_copy(rows_v, deg_s.at[idx_v], add=True)
        return c

    lax.fori_loop(0, _NBATCH, _step, 0)
    plsc.subcore_barrier()

    pltpu.sync_copy(
        deg_s.at[pl.ds(sub * _RPT, _RPT), :],
        deg_hbm.at[core, pl.ds(sub * _RPT, _RPT), :],
    )


@functools.partial(
    pl.kernel,
    out_type=jax.ShapeDtypeStruct((_NC, _NPAD, _D), jnp.float32),
    mesh=_sc_mesh,
    scratch_types=[
        pltpu.VMEM((_KB,), jnp.int32),
        pltpu.VMEM((_KB,), jnp.int32),
        pltpu.VMEM((_KB, _D), jnp.float32),
        pltpu.VMEM_SHARED((_NPAD, _D), jnp.float32),
        pltpu.SemaphoreType.DMA,
    ],
)
def _scatter_kernel(g_hbm, src_hbm, dst_hbm, agg_hbm, sidx, didx, rows_v, agg_s, sem):
    core = lax.axis_index("c")
    sub = lax.axis_index("s")
    tid = sub * _NC + core

    def _zrow(i, c):
        rows_v[i // (_D // 16), pl.ds((i % (_D // 16)) * 16, 16)] = jnp.zeros(
            (16,), jnp.float32)
        return c

    lax.fori_loop(0, _KB * (_D // 16), _zrow, 0)

    def _zero(r, c):
        pltpu.sync_copy(rows_v, agg_s.at[pl.ds(sub * _RPT + r * _KB, _KB), :])
        return c

    lax.fori_loop(0, _RPT // _KB, _zero, 0)
    plsc.subcore_barrier()

    def _step(i, c):
        base = tid * _EPW + i * _KB
        pltpu.sync_copy(src_hbm.at[pl.ds(base, _KB)], sidx)
        pltpu.sync_copy(dst_hbm.at[pl.ds(base, _KB)], didx)
        pltpu.async_copy(g_hbm.at[sidx], rows_v, sem).wait()
        pltpu.sync_copy(rows_v, agg_s.at[didx], add=True)
        return c

    lax.fori_loop(0, _NBATCH, _step, 0)
    plsc.subcore_barrier()

    pltpu.sync_copy(
        agg_s.at[pl.ds(sub * _RPT, _RPT), :],
        agg_hbm.at[core, pl.ds(sub * _RPT, _RPT), :],
    )


def _dot(a, b):
    return jnp.dot(a, b, preferred_element_type=jnp.float32,
                   precision=lax.Precision.HIGHEST)


def _pre_body(deg_ref, x_ref, w_ref, dis_ref, g_ref):
    d = deg_ref[0, :, 0:1] + deg_ref[1, :, 0:1] + 1.0
    dis = lax.rsqrt(d)
    dis_ref[...] = dis
    g_ref[...] = _dot(x_ref[...], w_ref[...]) * dis


_pre_call = pl.pallas_call(
    _pre_body,
    grid=(_NPAD // _BR,),
    in_specs=[
        pl.BlockSpec((_NC, _BR, _D), lambda i: (0, i, 0)),
        pl.BlockSpec((_BR, _D), lambda i: (i, 0)),
        pl.BlockSpec((_D, _D), lambda i: (0, 0)),
    ],
    out_specs=[
        pl.BlockSpec((_BR, 1), lambda i: (i, 0)),
        pl.BlockSpec((_BR, _D), lambda i: (i, 0)),
    ],
    out_shape=[
        jax.ShapeDtypeStruct((_NPAD, 1), jnp.float32),
        jax.ShapeDtypeStruct((_NPAD, _D), jnp.float32),
    ],
)


def _mid_body(agg_ref, g_ref, dis_ref, b_ref, w_ref, out_ref):
    dis = dis_ref[...]
    a = agg_ref[0] + agg_ref[1] + g_ref[...]
    h = jnp.maximum(dis * a + b_ref[...], 0.0)
    out_ref[...] = _dot(h, w_ref[...]) * dis


_mid_call = pl.pallas_call(
    _mid_body,
    grid=(_NPAD // _BR,),
    in_specs=[
        pl.BlockSpec((_NC, _BR, _D), lambda i: (0, i, 0)),
        pl.BlockSpec((_BR, _D), lambda i: (i, 0)),
        pl.BlockSpec((_BR, 1), lambda i: (i, 0)),
        pl.BlockSpec((1, _D), lambda i: (0, 0)),
        pl.BlockSpec((_D, _D), lambda i: (0, 0)),
    ],
    out_specs=pl.BlockSpec((_BR, _D), lambda i: (i, 0)),
    out_shape=jax.ShapeDtypeStruct((_NPAD, _D), jnp.float32),
)


def _fin_body(agg_ref, g_ref, dis_ref, b_ref, wl_ref, bl_ref, out_ref):
    dis = dis_ref[...]
    a = agg_ref[0] + agg_ref[1] + g_ref[...]
    h = jnp.maximum(dis * a + b_ref[...], 0.0)
    out_ref[...] = _dot(h, wl_ref[...]) + bl_ref[...]


def _make_fin(n_classes):
    return pl.pallas_call(
        _fin_body,
        grid=(_NPAD // _BR,),
        in_specs=[
            pl.BlockSpec((_NC, _BR, _D), lambda i: (0, i, 0)),
            pl.BlockSpec((_BR, _D), lambda i: (i, 0)),
            pl.BlockSpec((_BR, 1), lambda i: (i, 0)),
            pl.BlockSpec((1, _D), lambda i: (0, 0)),
            pl.BlockSpec((_D, n_classes), lambda i: (0, 0)),
            pl.BlockSpec((1, n_classes), lambda i: (0, 0)),
        ],
        out_specs=pl.BlockSpec((_BR, n_classes), lambda i: (i, 0)),
        out_shape=jax.ShapeDtypeStruct((_NPAD, n_classes), jnp.float32),
    )


def kernel(x, edge_index, W1, b1, W2, b2, W3, b3, Wl, bl):
    src = edge_index[0].astype(jnp.int32)
    dst = edge_index[1].astype(jnp.int32)
    pad_e = _EPAD - src.shape[0]
    src = jnp.pad(src, (0, pad_e), constant_values=_N)
    dst = jnp.pad(dst, (0, pad_e), constant_values=_N)
    xp = jnp.pad(x, ((0, _NPAD - x.shape[0]), (0, 0)))

    deg = _deg_kernel(dst)
    dis, g1 = _pre_call(deg, xp, W1)
    agg1 = _scatter_kernel(g1, src, dst)
    g2 = _mid_call(agg1, g1, dis, b1.reshape(1, -1), W2)
    agg2 = _scatter_kernel(g2, src, dst)
    g3 = _mid_call(agg2, g2, dis, b2.reshape(1, -1), W3)
    agg3 = _scatter_kernel(g3, src, dst)
    out = _make_fin(Wl.shape[1])(agg3, g3, dis, b3.reshape(1, -1), Wl,
                                 bl.reshape(1, -1))
    return out[:x.shape[0]]
